# Initial kernel scaffold; baseline (speedup 1.0000x reference)
import jax
import jax.numpy as jnp
from jax.experimental import pallas as pl


def kernel(x, flows, W1, g1, bt1, W2, g2, bt2, W3, b3):
    raise NotImplementedError("write your pallas kernel here")


# speedup vs baseline: 16.0442x; 16.0442x over previous
import jax
import jax.numpy as jnp
from jax import lax
from jax.experimental import pallas as pl
from jax.experimental.pallas import tpu as pltpu

HH = 96
R = 7
WS = 15
K = 7
GH = HH + 2 * R
GW = 128
NF2 = (GH + 1) * GW
NS = HH * GW
DC = R * GW + R
NOFF = WS * WS
NOFF_PAD = 232
NB = 4
NL = (HH // NB) * GW
NW = (HH // NB + WS - 1) * GW


def _flatgrid(img, cv):
    b, c = img.shape[0], img.shape[1]
    p = jnp.pad(img, ((0, 0), (0, 0), (R, R + 1), (R, GW - HH - R)),
                constant_values=cv)
    return p.reshape(b, c, NF2)


def _unjunk(flat, c):
    return flat.reshape(flat.shape[0], c, HH, GW)[..., :HH]


def _conv_body(relu, co):
    def body(src_ref, w_ref, s_ref, b_ref, out_ref):
        for band in range(NB):
            base = band * NL
            acc = jnp.zeros((co, NL), jnp.float32)
            for dyp in range(3):
                for dxp in range(3):
                    off = (6 + dyp) * GW + 6 + dxp + base
                    acc = acc + jnp.dot(w_ref[dyp, dxp],
                                        src_ref[0, :, off:off + NL],
                                        preferred_element_type=jnp.float32)
            acc = acc * s_ref[...] + b_ref[...]
            if relu:
                acc = jnp.maximum(acc, 0.0)
            out_ref[0, :, base:base + NL] = acc
    return body


def _conv_call(src, wt, sv, bv, co, relu):
    b, ci = src.shape[0], src.shape[1]
    return pl.pallas_call(
        _conv_body(relu, co),
        grid=(b,),
        in_specs=[
            pl.BlockSpec((1, ci, NF2), lambda i: (i, 0, 0)),
            pl.BlockSpec((3, 3, co, ci), lambda i: (0, 0, 0, 0)),
            pl.BlockSpec((co, 1), lambda i: (0, 0)),
            pl.BlockSpec((co, 1), lambda i: (0, 0)),
        ],
        out_specs=pl.BlockSpec((1, co, NS), lambda i: (i, 0, 0)),
        out_shape=jax.ShapeDtypeStruct((b, co, NS), jnp.float32),
    )(src.astype(jnp.bfloat16), wt.astype(jnp.bfloat16), sv, bv)


def _knn_body(ep_ref, xp_ref, out_ref, es, xs, ds):
    iota_off = lax.broadcasted_iota(jnp.int32, (NOFF_PAD, 1), 0)
    iota16 = lax.broadcasted_iota(jnp.int32, (16, 1), 0)
    for r in range(NOFF, NOFF_PAD):
        ds[r:r + 1, :] = jnp.full((1, NL), 1e30, jnp.float32)

    for band in range(NB):
        base = band * NL
        ecb = ep_ref[0, :, base + DC:base + DC + NL]

        for dx in range(WS):
            es[...] = ep_ref[0, :, base + dx:base + dx + NW]
            for dy in range(WS):
                d = es[:, dy * GW:dy * GW + NL] - ecb
                ds[dy * WS + dx:dy * WS + dx + 1, :] = jnp.sum(
                    d * d, axis=0, keepdims=True)

        dvals, dargs = [], []
        for k in range(K):
            dcur = ds[...]
            m = jnp.min(dcur, axis=0, keepdims=True)
            cand = jnp.where(dcur == m, iota_off, NOFF_PAD)
            arg = jnp.min(cand, axis=0, keepdims=True)
            dvals.append(m)
            dargs.append(arg)
            if k < K - 1:
                ds[...] = jnp.where(iota_off == arg, 1e30, dcur)

        wraw = [jnp.exp(dvals[0] - dv) for dv in dvals]
        inv = 1.0 / sum(wraw)
        wnorm = [w * inv for w in wraw]

        agg = jnp.zeros((64, NL), jnp.float32)
        for dx in range(WS):
            xs[...] = xp_ref[0, :, base + dx:base + dx + NW]
            offs = iota16 * WS + dx
            wm = jnp.zeros((16, NL), jnp.float32)
            for k in range(K):
                wm = wm + jnp.where(dargs[k] == offs, wnorm[k], 0.0)
            for dy in range(WS):
                agg = agg + wm[dy:dy + 1, :] * xs[:, dy * GW:dy * GW + NL]
        out_ref[0, :, base:base + NL] = agg


def kernel(x, flows, W1, g1, bt1, W2, g2, bt2, W3, b3):
    del flows
    B = x.shape[0]
    s1v = (g1 / jnp.sqrt(1.0 + 1e-5)).reshape(64, 1)
    s2v = (g2 / jnp.sqrt(1.0 + 1e-5)).reshape(64, 1)
    s3v = jnp.ones((8, 1), jnp.float32)
    w1t = jnp.transpose(W1, (2, 3, 0, 1))
    w2t = jnp.transpose(W2, (2, 3, 0, 1))
    w3t = jnp.transpose(W3, (2, 3, 0, 1))
    b1v = bt1.reshape(64, 1)
    b2v = bt2.reshape(64, 1)
    b3v = b3.reshape(8, 1)

    xg = _flatgrid(x, 0.0)
    h1 = _conv_call(xg, w1t, s1v, b1v, 64, True)
    h2 = _conv_call(_flatgrid(_unjunk(h1, 64), 0.0), w2t, s2v, b2v, 64, True)
    e = _conv_call(_flatgrid(_unjunk(h2, 64), 0.0), w3t, s3v, b3v, 8, False)
    eg = _flatgrid(_unjunk(e, 8), 1e4)

    agg = pl.pallas_call(
        _knn_body,
        grid=(B,),
        in_specs=[
            pl.BlockSpec((1, 8, NF2), lambda i: (i, 0, 0)),
            pl.BlockSpec((1, 64, NF2), lambda i: (i, 0, 0)),
        ],
        out_specs=pl.BlockSpec((1, 64, NS), lambda i: (i, 0, 0)),
        out_shape=jax.ShapeDtypeStruct((B, 64, NS), jnp.float32),
        scratch_shapes=[
            pltpu.VMEM((8, NW), jnp.float32),
            pltpu.VMEM((64, NW), jnp.float32),
            pltpu.VMEM((NOFF_PAD, NL), jnp.float32),
        ],
    )(eg, xg)

    return jnp.concatenate([x, _unjunk(agg, 64)], axis=1)
